# baseline (speedup 1.0000x reference)
import functools

import jax
import jax.numpy as jnp
from jax import lax
from jax.experimental import pallas as pl
from jax.experimental.pallas import tpu as pltpu
from jax.experimental.pallas import tpu_sc as plsc

M = 8
KS = 256
DS = 16

NC = 2
NS = 16
NW = NC * NS


def _encode_body(vecs_ref, cwt_ref, codes_ref):
    def _score(m):
        sub = vecs_ref[:, m * DS:(m + 1) * DS]
        cwt = cwt_ref[m * DS:(m + 1) * DS, :]
        xc = jnp.dot(sub, cwt, preferred_element_type=jnp.float32)
        c2h = 0.5 * jnp.sum(cwt * cwt, axis=0, keepdims=True)
        return c2h - xc

    def _argmin(score, m):
        s0 = score[:, :128]
        s1 = score[:, 128:]
        m0 = jnp.min(s0, axis=1, keepdims=True)
        i0 = jnp.argmin(s0, axis=1, keepdims=True).astype(jnp.int32)
        m1 = jnp.min(s1, axis=1, keepdims=True)
        i1 = jnp.argmin(s1, axis=1, keepdims=True).astype(jnp.int32)
        idx = jnp.where(m1 < m0, i1 + 128, i0)
        return idx + m * KS

    cols = []
    score_prev = _score(0)
    for m in range(1, M):
        score_next = _score(m)
        cols.append(_argmin(score_prev, m - 1))
        score_prev = score_next
    cols.append(_argmin(score_prev, M - 1))
    codes_ref[:, :] = jnp.concatenate(cols, axis=1)


def _encode(vecs, cwt, block_b):
    n = vecs.shape[0]
    grid = (n // block_b,)
    return pl.pallas_call(
        _encode_body,
        grid=grid,
        in_specs=[
            pl.BlockSpec((block_b, M * DS), lambda i: (i, 0)),
            pl.BlockSpec((M * DS, KS), lambda i: (0, 0)),
        ],
        out_specs=pl.BlockSpec((block_b, M), lambda i: (i, 0)),
        out_shape=jax.ShapeDtypeStruct((n, M), jnp.int32),
    )(vecs, cwt)


def _make_decode(total, chunk):
    b_per_w = total // NW
    n_chunks = b_per_w // chunk
    mesh = plsc.VectorSubcoreMesh(
        core_axis_name="c", subcore_axis_name="s",
        num_cores=NC, num_subcores=NS)

    @functools.partial(
        pl.kernel,
        out_type=jax.ShapeDtypeStruct((total, DS), jnp.float32),
        mesh=mesh,
        scratch_types=[
            pltpu.VMEM((chunk,), jnp.int32),
            pltpu.VMEM((chunk, DS), jnp.float32),
            pltpu.SemaphoreType.DMA,
        ],
        compiler_params=pltpu.CompilerParams(use_tc_tiling_on_sc=False),
    )
    def decode(table_hbm, idx_hbm, out_hbm, idx_v, rows_v, sem):
        wid = lax.axis_index("s") * NC + lax.axis_index("c")
        base = wid * b_per_w
        for c in range(n_chunks):
            off = base + c * chunk
            pltpu.sync_copy(idx_hbm.at[pl.ds(off, chunk)], idx_v)
            pltpu.async_copy(table_hbm.at[idx_v], rows_v, sem).wait()
            pltpu.sync_copy(rows_v, out_hbm.at[pl.ds(off, chunk)])

    return decode


def kernel(vecs, codewords):
    n, d = vecs.shape
    m_, ks_, ds_ = codewords.shape
    cwt = codewords.transpose(0, 2, 1).reshape(m_ * ds_, ks_)
    table = codewords.reshape(m_ * ks_, ds_)
    codes = _encode(vecs, cwt, block_b=4096)
    flat_codes = codes.reshape(n * m_)
    rows = _make_decode(n * m_, 2048)(table, flat_codes)
    return rows.reshape(n, d)


# speedup vs baseline: 1.8303x; 1.8303x over previous
import functools

import jax
import jax.numpy as jnp
from jax import lax
from jax.experimental import pallas as pl
from jax.experimental.pallas import tpu as pltpu
from jax.experimental.pallas import tpu_sc as plsc

M = 8
KS = 256
DS = 16

NC = 2
NS = 16
NW = NC * NS


def _encode_body(vecst_ref, cw_ref, codes_ref):
    def _score(m):
        sub = vecst_ref[m * DS:(m + 1) * DS, :]
        cw = cw_ref[m * KS:(m + 1) * KS, :]
        xc = jnp.dot(cw, sub, preferred_element_type=jnp.float32)
        c2h = 0.5 * jnp.sum(cw * cw, axis=1, keepdims=True)
        return c2h - xc

    def _argmin(score, m):
        idx = jnp.argmin(score, axis=0, keepdims=True).astype(jnp.int32)
        return idx + m * KS

    rows = []
    score_prev = _score(0)
    for m in range(1, M):
        score_next = _score(m)
        rows.append(_argmin(score_prev, m - 1))
        score_prev = score_next
    rows.append(_argmin(score_prev, M - 1))
    codes_ref[:, :] = jnp.concatenate(rows, axis=0)


def _encode(vecst, cw, block_b):
    n = vecst.shape[1]
    grid = (n // block_b,)
    return pl.pallas_call(
        _encode_body,
        grid=grid,
        in_specs=[
            pl.BlockSpec((M * DS, block_b), lambda i: (0, i)),
            pl.BlockSpec((M * KS, DS), lambda i: (0, 0)),
        ],
        out_specs=pl.BlockSpec((M, block_b), lambda i: (0, i)),
        out_shape=jax.ShapeDtypeStruct((M, n), jnp.int32),
    )(vecst, cw)


def _make_decode(total, chunk):
    b_per_w = total // NW
    n_chunks = b_per_w // chunk
    mesh = plsc.VectorSubcoreMesh(
        core_axis_name="c", subcore_axis_name="s",
        num_cores=NC, num_subcores=NS)

    @functools.partial(
        pl.kernel,
        out_type=jax.ShapeDtypeStruct((total, DS), jnp.float32),
        mesh=mesh,
        scratch_types=[
            pltpu.VMEM((chunk,), jnp.int32),
            pltpu.VMEM((chunk, DS), jnp.float32),
            pltpu.SemaphoreType.DMA,
        ],
        compiler_params=pltpu.CompilerParams(use_tc_tiling_on_sc=False),
    )
    def decode(table_hbm, idx_hbm, out_hbm, idx_v, rows_v, sem):
        wid = lax.axis_index("s") * NC + lax.axis_index("c")
        base = wid * b_per_w
        for c in range(n_chunks):
            off = base + c * chunk
            pltpu.sync_copy(idx_hbm.at[pl.ds(off, chunk)], idx_v)
            pltpu.async_copy(table_hbm.at[idx_v], rows_v, sem).wait()
            pltpu.sync_copy(rows_v, out_hbm.at[pl.ds(off, chunk)])

    return decode


def kernel(vecs, codewords):
    n, d = vecs.shape
    m_, ks_, ds_ = codewords.shape
    table = codewords.reshape(m_ * ks_, ds_)
    codes = _encode(vecs.T, table, block_b=4096)
    flat_codes = codes.T.reshape(n * m_)
    rows = _make_decode(n * m_, 2048)(table, flat_codes)
    return rows.reshape(n, d)
